# Initial kernel scaffold; baseline (speedup 1.0000x reference)
import jax
import jax.numpy as jnp
from jax.experimental import pallas as pl


def kernel(x, edge_index, W1, b1, W2, b2):
    raise NotImplementedError("write your pallas kernel here")


# speedup vs baseline: 20.9826x; 20.9826x over previous
import functools

import jax
import jax.numpy as jnp
from jax import lax
from jax.experimental import pallas as pl
from jax.experimental.pallas import tpu as pltpu
from jax.experimental.pallas import tpu_sc as plsc

N = 10000
E = 320000
D = 128

NC = 2
NS = 16
NW = NC * NS
EC = 128
KJ = 80
EPAD = NW * EC * KJ
NP = 10240
RPW = NP // NS
BLK = 1024


def _mesh():
    return plsc.VectorSubcoreMesh(core_axis_name="c", subcore_axis_name="s")


@functools.partial(
    pl.kernel,
    mesh=_mesh(),
    out_type=jax.ShapeDtypeStruct((NC, NP, 16), jnp.float32),
    scratch_types=[
        pltpu.VMEM((KJ, EC), jnp.int32),
        pltpu.VMEM((EC, 16), jnp.float32),
        pltpu.VMEM_SHARED((NP, 16), jnp.float32),
    ],
)
def _sc_deg(dst_hbm, z_hbm, ones_hbm, out_hbm, dst_v, ones_v, acc):
    c = lax.axis_index("c")
    s = lax.axis_index("s")
    w = s * NC + c
    pltpu.sync_copy(z_hbm, acc.at[pl.ds(s * RPW, RPW)])
    pltpu.sync_copy(ones_hbm, ones_v)
    pltpu.sync_copy(dst_hbm.at[pl.ds(w * KJ, KJ)], dst_v)
    plsc.subcore_barrier()

    def body(j, carry):
        pltpu.sync_copy(ones_v, acc.at[dst_v.at[j]], add=True)
        return carry

    lax.fori_loop(0, KJ, body, 0)
    plsc.subcore_barrier()
    pltpu.sync_copy(acc.at[pl.ds(s * RPW, RPW)],
                    out_hbm.at[c, pl.ds(s * RPW, RPW)])


@functools.partial(
    pl.kernel,
    mesh=_mesh(),
    out_type=jax.ShapeDtypeStruct((NC, NP, D), jnp.float32),
    scratch_types=[
        pltpu.VMEM((KJ, EC), jnp.int32),
        pltpu.VMEM((KJ, EC), jnp.int32),
        pltpu.VMEM((EC, D), jnp.float32),
        pltpu.VMEM_SHARED((NP, D), jnp.float32),
        pltpu.SemaphoreType.DMA,
    ],
)
def _sc_agg(y_hbm, z_hbm, src_hbm, dst_hbm, out_hbm,
            src_v, dst_v, rows_v, acc, sem):
    c = lax.axis_index("c")
    s = lax.axis_index("s")
    w = s * NC + c
    for kk in range(RPW // EC):
        pltpu.sync_copy(z_hbm, acc.at[pl.ds(s * RPW + kk * EC, EC)])
    pltpu.sync_copy(src_hbm.at[pl.ds(w * KJ, KJ)], src_v)
    pltpu.sync_copy(dst_hbm.at[pl.ds(w * KJ, KJ)], dst_v)
    plsc.subcore_barrier()

    def body(j, carry):
        pltpu.async_copy(y_hbm.at[src_v.at[j]], rows_v, sem).wait()
        pltpu.sync_copy(rows_v, acc.at[dst_v.at[j]], add=True)
        return carry

    lax.fori_loop(0, KJ, body, 0)
    plsc.subcore_barrier()
    pltpu.sync_copy(acc.at[pl.ds(s * RPW, RPW)],
                    out_hbm.at[c, pl.ds(s * RPW, RPW)])


def _dis(deg_ref):
    return lax.rsqrt(deg_ref[0, :, 0:1] + deg_ref[1, :, 0:1] + 1.0)


def _tc1_body(deg_ref, x_ref, w_ref, y_ref):
    y_ref[...] = jnp.dot(x_ref[...], w_ref[...],
                         preferred_element_type=jnp.float32) * _dis(deg_ref)


def _tc2_body(deg_ref, a_ref, y_ref, b_ref, w_ref, o_ref):
    dis = _dis(deg_ref)
    pre = (a_ref[0] + a_ref[1] + y_ref[...]) * dis + b_ref[...]
    h = jnp.maximum(pre, 0.0)
    o_ref[...] = jnp.dot(h, w_ref[...],
                         preferred_element_type=jnp.float32) * dis


def _tc3_body(deg_ref, a_ref, y_ref, b_ref, o_ref):
    pre = (a_ref[0] + a_ref[1] + y_ref[...]) * _dis(deg_ref) + b_ref[...]
    o_ref[...] = jnp.maximum(pre, 0.0)


_DEG_SPEC = pl.BlockSpec((NC, BLK, 16), lambda i: (0, i, 0))
_AGG_SPEC = pl.BlockSpec((NC, BLK, D), lambda i: (0, i, 0))
_ROW_SPEC = pl.BlockSpec((BLK, D), lambda i: (i, 0))
_W_SPEC = pl.BlockSpec((D, D), lambda i: (0, 0))
_B_SPEC = pl.BlockSpec((1, D), lambda i: (0, 0))
_OUT = jax.ShapeDtypeStruct((NP, D), jnp.float32)
_GRID = (NP // BLK,)


def _tc1(deg2, xp, W1):
    return pl.pallas_call(
        _tc1_body, grid=_GRID,
        in_specs=[_DEG_SPEC, _ROW_SPEC, _W_SPEC],
        out_specs=_ROW_SPEC, out_shape=_OUT)(deg2, xp, W1)


def _tc2(deg2, agg, y, b, W):
    return pl.pallas_call(
        _tc2_body, grid=_GRID,
        in_specs=[_DEG_SPEC, _AGG_SPEC, _ROW_SPEC, _B_SPEC, _W_SPEC],
        out_specs=_ROW_SPEC, out_shape=_OUT)(deg2, agg, y, b, W)


def _tc3(deg2, agg, y, b):
    return pl.pallas_call(
        _tc3_body, grid=_GRID,
        in_specs=[_DEG_SPEC, _AGG_SPEC, _ROW_SPEC, _B_SPEC],
        out_specs=_ROW_SPEC, out_shape=_OUT)(deg2, agg, y, b)


def kernel(x, edge_index, W1, b1, W2, b2):
    src = edge_index[0]
    dst = edge_index[1]
    padi = (N + jnp.arange(EPAD - E, dtype=jnp.int32) % (NP - N)).astype(jnp.int32)
    srcp = jnp.concatenate([src, padi]).reshape(EPAD // EC, EC)
    dstp = jnp.concatenate([dst, padi]).reshape(EPAD // EC, EC)
    xp = jnp.pad(x, ((0, NP - N), (0, 0)))
    z16 = jnp.zeros((RPW, 16), jnp.float32)
    z128 = jnp.zeros((EC, D), jnp.float32)
    ones16 = jnp.ones((EC, 16), jnp.float32)

    deg2 = _sc_deg(dstp, z16, ones16)
    y1 = _tc1(deg2, xp, W1)
    agg1 = _sc_agg(y1, z128, srcp, dstp)
    y2 = _tc2(deg2, agg1, y1, b1.reshape(1, D), W2)
    agg2 = _sc_agg(y2, z128, srcp, dstp)
    out = _tc3(deg2, agg2, y2, b2.reshape(1, D))
    return out[:N]


# speedup vs baseline: 26.1567x; 1.2466x over previous
import functools

import jax
import jax.numpy as jnp
from jax import lax
from jax.experimental import pallas as pl
from jax.experimental.pallas import tpu as pltpu
from jax.experimental.pallas import tpu_sc as plsc

N = 10000
E = 320000
D = 128

NC = 2
NS = 16
NW = NC * NS
EC = 128
KJ = 80
KJH = 40
EPAD = NW * EC * KJ
NP = 10240
RPW = NP // NS
BLK = 1024


def _mesh():
    return plsc.VectorSubcoreMesh(core_axis_name="c", subcore_axis_name="s")


@functools.partial(
    pl.kernel,
    mesh=_mesh(),
    out_type=jax.ShapeDtypeStruct((NC, NP, 8), jnp.float32),
    scratch_types=[
        pltpu.VMEM((KJ, EC), jnp.int32),
        pltpu.VMEM((EC, 8), jnp.float32),
        pltpu.VMEM_SHARED((NP, 8), jnp.float32),
    ],
)
def _sc_deg(dst_hbm, z_hbm, ones_hbm, out_hbm, dst_v, ones_v, acc):
    c = lax.axis_index("c")
    s = lax.axis_index("s")
    w = s * NC + c
    pltpu.sync_copy(z_hbm, acc.at[pl.ds(s * RPW, RPW)])
    pltpu.sync_copy(ones_hbm, ones_v)
    pltpu.sync_copy(dst_hbm.at[pl.ds(w * KJ, KJ)], dst_v)
    plsc.subcore_barrier()

    def body(j, carry):
        pltpu.sync_copy(ones_v, acc.at[dst_v.at[j]], add=True)
        return carry

    lax.fori_loop(0, KJ, body, 0)
    plsc.subcore_barrier()
    pltpu.sync_copy(acc.at[pl.ds(s * RPW, RPW)],
                    out_hbm.at[c, pl.ds(s * RPW, RPW)])


@functools.partial(
    pl.kernel,
    mesh=_mesh(),
    out_type=jax.ShapeDtypeStruct((NC, NP, D), jnp.float32),
    scratch_types=[
        pltpu.VMEM((KJH, EC), jnp.int32),
        pltpu.VMEM((KJH, EC), jnp.int32),
        pltpu.VMEM((2, EC, D), jnp.float32),
        pltpu.VMEM_SHARED((NP, D), jnp.float32),
        pltpu.SemaphoreType.DMA,
        pltpu.SemaphoreType.DMA,
    ],
)
def _sc_agg(y_hbm, z_hbm, src_hbm, dst_hbm, out_hbm,
            src_v, dst_v, rows_v, acc, sem_g, sem_s):
    c = lax.axis_index("c")
    s = lax.axis_index("s")
    w = s * NC + c
    for kk in range(RPW // EC):
        pltpu.sync_copy(z_hbm, acc.at[pl.ds(s * RPW + kk * EC, EC)])
    plsc.subcore_barrier()

    for q in range(KJ // KJH):
        base = w * KJ + q * KJH
        pltpu.sync_copy(src_hbm.at[pl.ds(base, KJH)], src_v)
        pltpu.sync_copy(dst_hbm.at[pl.ds(base, KJH)], dst_v)
        pltpu.async_copy(y_hbm.at[src_v.at[0]], rows_v.at[0], sem_g)

        def body(j, carry):
            b = lax.rem(j, 2)
            nb = lax.rem(j + 1, 2)
            pltpu.make_async_copy(y_hbm.at[src_v.at[j]], rows_v.at[b],
                                  sem_g).wait()

            @pl.when(j >= 1)
            def _wait_prev_scatter():
                pltpu.make_async_copy(rows_v.at[nb], acc.at[dst_v.at[j - 1]],
                                      sem_s).wait()

            @pl.when(j < KJH - 1)
            def _start_next_gather():
                pltpu.async_copy(y_hbm.at[src_v.at[j + 1]], rows_v.at[nb],
                                 sem_g)

            pltpu.async_copy(rows_v.at[b], acc.at[dst_v.at[j]], sem_s,
                             add=True)
            return carry

        lax.fori_loop(0, KJH, body, 0)
        pltpu.make_async_copy(rows_v.at[(KJH - 1) % 2],
                              acc.at[dst_v.at[KJH - 1]], sem_s).wait()
    plsc.subcore_barrier()
    pltpu.sync_copy(acc.at[pl.ds(s * RPW, RPW)],
                    out_hbm.at[c, pl.ds(s * RPW, RPW)])


def _dis(deg_ref):
    return lax.rsqrt(deg_ref[0, :, 0:1] + deg_ref[1, :, 0:1] + 1.0)


def _tc1_body(deg_ref, x_ref, w_ref, y_ref):
    y_ref[...] = jnp.dot(x_ref[...], w_ref[...],
                         preferred_element_type=jnp.float32) * _dis(deg_ref)


def _tc2_body(deg_ref, a_ref, y_ref, b_ref, w_ref, o_ref):
    dis = _dis(deg_ref)
    pre = (a_ref[0] + a_ref[1] + y_ref[...]) * dis + b_ref[...]
    h = jnp.maximum(pre, 0.0)
    o_ref[...] = jnp.dot(h, w_ref[...],
                         preferred_element_type=jnp.float32) * dis


def _tc3_body(deg_ref, a_ref, y_ref, b_ref, o_ref):
    pre = (a_ref[0] + a_ref[1] + y_ref[...]) * _dis(deg_ref) + b_ref[...]
    o_ref[...] = jnp.maximum(pre, 0.0)


_DEG_SPEC = pl.BlockSpec((NC, BLK, 8), lambda i: (0, i, 0))
_AGG_SPEC = pl.BlockSpec((NC, BLK, D), lambda i: (0, i, 0))
_ROW_SPEC = pl.BlockSpec((BLK, D), lambda i: (i, 0))
_W_SPEC = pl.BlockSpec((D, D), lambda i: (0, 0))
_B_SPEC = pl.BlockSpec((1, D), lambda i: (0, 0))
_OUT = jax.ShapeDtypeStruct((NP, D), jnp.float32)
_GRID = (NP // BLK,)


def _tc1(deg2, xp, W1):
    return pl.pallas_call(
        _tc1_body, grid=_GRID,
        in_specs=[_DEG_SPEC, _ROW_SPEC, _W_SPEC],
        out_specs=_ROW_SPEC, out_shape=_OUT)(deg2, xp, W1)


def _tc2(deg2, agg, y, b, W):
    return pl.pallas_call(
        _tc2_body, grid=_GRID,
        in_specs=[_DEG_SPEC, _AGG_SPEC, _ROW_SPEC, _B_SPEC, _W_SPEC],
        out_specs=_ROW_SPEC, out_shape=_OUT)(deg2, agg, y, b, W)


def _tc3(deg2, agg, y, b):
    return pl.pallas_call(
        _tc3_body, grid=_GRID,
        in_specs=[_DEG_SPEC, _AGG_SPEC, _ROW_SPEC, _B_SPEC],
        out_specs=_ROW_SPEC, out_shape=_OUT)(deg2, agg, y, b)


def kernel(x, edge_index, W1, b1, W2, b2):
    src = edge_index[0]
    dst = edge_index[1]
    padi = (N + jnp.arange(EPAD - E, dtype=jnp.int32) % (NP - N)).astype(jnp.int32)
    srcp = jnp.concatenate([src, padi]).reshape(EPAD // EC, EC)
    dstp = jnp.concatenate([dst, padi]).reshape(EPAD // EC, EC)
    xp = jnp.pad(x, ((0, NP - N), (0, 0)))
    z16 = jnp.zeros((RPW, 8), jnp.float32)
    z128 = jnp.zeros((EC, D), jnp.float32)
    ones16 = jnp.ones((EC, 8), jnp.float32)

    deg2 = _sc_deg(dstp, z16, ones16)
    y1 = _tc1(deg2, xp, W1)
    agg1 = _sc_agg(y1, z128, srcp, dstp)
    y2 = _tc2(deg2, agg1, y1, b1.reshape(1, D), W2)
    agg2 = _sc_agg(y2, z128, srcp, dstp)
    out = _tc3(deg2, agg2, y2, b2.reshape(1, D))
    return out[:N]


# speedup vs baseline: 26.3550x; 1.0076x over previous
import functools

import jax
import jax.numpy as jnp
from jax import lax
from jax.experimental import pallas as pl
from jax.experimental.pallas import tpu as pltpu
from jax.experimental.pallas import tpu_sc as plsc

N = 10000
E = 320000
D = 128

NC = 2
NS = 16
NW = NC * NS
EC = 128
KJ = 80
KJH = 40
EPAD = NW * EC * KJ
NP = 10240
RPW = NP // NS
BLK = 1024


def _mesh():
    return plsc.VectorSubcoreMesh(core_axis_name="c", subcore_axis_name="s")


@functools.partial(
    pl.kernel,
    mesh=_mesh(),
    out_type=jax.ShapeDtypeStruct((NC, NP, 8), jnp.float32),
    scratch_types=[
        pltpu.VMEM((KJ, EC), jnp.int32),
        pltpu.VMEM((EC, 8), jnp.float32),
        pltpu.VMEM_SHARED((NP, 8), jnp.float32),
        pltpu.SemaphoreType.DMA,
    ],
)
def _sc_deg(dst_hbm, z_hbm, ones_hbm, out_hbm, dst_v, ones_v, acc, sem):
    c = lax.axis_index("c")
    s = lax.axis_index("s")
    w = s * NC + c
    pltpu.sync_copy(z_hbm, acc.at[pl.ds(s * RPW, RPW)])
    pltpu.sync_copy(ones_hbm, ones_v)
    pltpu.sync_copy(dst_hbm.at[pl.ds(w * KJ, KJ)], dst_v)
    plsc.subcore_barrier()

    def fire(j, carry):
        pltpu.async_copy(ones_v, acc.at[dst_v.at[j]], sem, add=True)
        return carry

    lax.fori_loop(0, KJ, fire, 0)

    def drain(j, carry):
        pltpu.make_async_copy(ones_v, acc.at[dst_v.at[0]], sem).wait()
        return carry

    lax.fori_loop(0, KJ, drain, 0)
    plsc.subcore_barrier()
    pltpu.sync_copy(acc.at[pl.ds(s * RPW, RPW)],
                    out_hbm.at[c, pl.ds(s * RPW, RPW)])


@functools.partial(
    pl.kernel,
    mesh=_mesh(),
    out_type=jax.ShapeDtypeStruct((NC, NP, D), jnp.float32),
    scratch_types=[
        pltpu.VMEM((KJH, EC), jnp.int32),
        pltpu.VMEM((KJH, EC), jnp.int32),
        pltpu.VMEM((2, EC, D), jnp.float32),
        pltpu.VMEM_SHARED((NP, D), jnp.float32),
        pltpu.SemaphoreType.DMA,
        pltpu.SemaphoreType.DMA,
    ],
)
def _sc_agg(y_hbm, z_hbm, src_hbm, dst_hbm, out_hbm,
            src_v, dst_v, rows_v, acc, sem_g, sem_s):
    c = lax.axis_index("c")
    s = lax.axis_index("s")
    w = s * NC + c
    for kk in range(RPW // EC):
        pltpu.async_copy(z_hbm, acc.at[pl.ds(s * RPW + kk * EC, EC)], sem_s)
    for kk in range(RPW // EC):
        pltpu.make_async_copy(z_hbm, acc.at[pl.ds(s * RPW, EC)], sem_s).wait()
    plsc.subcore_barrier()

    for q in range(KJ // KJH):
        base = w * KJ + q * KJH
        pltpu.sync_copy(src_hbm.at[pl.ds(base, KJH)], src_v)
        pltpu.sync_copy(dst_hbm.at[pl.ds(base, KJH)], dst_v)
        pltpu.async_copy(y_hbm.at[src_v.at[0]], rows_v.at[0], sem_g)

        def body(j, carry):
            b = lax.rem(j, 2)
            nb = lax.rem(j + 1, 2)
            pltpu.make_async_copy(y_hbm.at[src_v.at[j]], rows_v.at[b],
                                  sem_g).wait()

            @pl.when(j >= 1)
            def _wait_prev_scatter():
                pltpu.make_async_copy(rows_v.at[nb], acc.at[dst_v.at[j - 1]],
                                      sem_s).wait()

            @pl.when(j < KJH - 1)
            def _start_next_gather():
                pltpu.async_copy(y_hbm.at[src_v.at[j + 1]], rows_v.at[nb],
                                 sem_g)

            pltpu.async_copy(rows_v.at[b], acc.at[dst_v.at[j]], sem_s,
                             add=True)
            return carry

        lax.fori_loop(0, KJH, body, 0)
        pltpu.make_async_copy(rows_v.at[(KJH - 1) % 2],
                              acc.at[dst_v.at[KJH - 1]], sem_s).wait()
    plsc.subcore_barrier()
    pltpu.sync_copy(acc.at[pl.ds(s * RPW, RPW)],
                    out_hbm.at[c, pl.ds(s * RPW, RPW)])


def _dis(deg_ref):
    return lax.rsqrt(deg_ref[0, :, 0:1] + deg_ref[1, :, 0:1] + 1.0)


def _tc1_body(deg_ref, x_ref, w_ref, y_ref):
    y_ref[...] = jnp.dot(x_ref[...], w_ref[...],
                         preferred_element_type=jnp.float32) * _dis(deg_ref)


def _tc2_body(deg_ref, a_ref, y_ref, b_ref, w_ref, o_ref):
    dis = _dis(deg_ref)
    pre = (a_ref[0] + a_ref[1] + y_ref[...]) * dis + b_ref[...]
    h = jnp.maximum(pre, 0.0)
    o_ref[...] = jnp.dot(h, w_ref[...],
                         preferred_element_type=jnp.float32) * dis


def _tc3_body(deg_ref, a_ref, y_ref, b_ref, o_ref):
    pre = (a_ref[0] + a_ref[1] + y_ref[...]) * _dis(deg_ref) + b_ref[...]
    o_ref[...] = jnp.maximum(pre, 0.0)


_DEG_SPEC = pl.BlockSpec((NC, BLK, 8), lambda i: (0, i, 0))
_AGG_SPEC = pl.BlockSpec((NC, BLK, D), lambda i: (0, i, 0))
_ROW_SPEC = pl.BlockSpec((BLK, D), lambda i: (i, 0))
_W_SPEC = pl.BlockSpec((D, D), lambda i: (0, 0))
_B_SPEC = pl.BlockSpec((1, D), lambda i: (0, 0))
_OUT = jax.ShapeDtypeStruct((NP, D), jnp.float32)
_GRID = (NP // BLK,)


def _tc1(deg2, xp, W1):
    return pl.pallas_call(
        _tc1_body, grid=_GRID,
        in_specs=[_DEG_SPEC, _ROW_SPEC, _W_SPEC],
        out_specs=_ROW_SPEC, out_shape=_OUT)(deg2, xp, W1)


def _tc2(deg2, agg, y, b, W):
    return pl.pallas_call(
        _tc2_body, grid=_GRID,
        in_specs=[_DEG_SPEC, _AGG_SPEC, _ROW_SPEC, _B_SPEC, _W_SPEC],
        out_specs=_ROW_SPEC, out_shape=_OUT)(deg2, agg, y, b, W)


def _tc3(deg2, agg, y, b):
    return pl.pallas_call(
        _tc3_body, grid=_GRID,
        in_specs=[_DEG_SPEC, _AGG_SPEC, _ROW_SPEC, _B_SPEC],
        out_specs=_ROW_SPEC, out_shape=_OUT)(deg2, agg, y, b)


def kernel(x, edge_index, W1, b1, W2, b2):
    src = edge_index[0]
    dst = edge_index[1]
    padi = (N + jnp.arange(EPAD - E, dtype=jnp.int32) % (NP - N)).astype(jnp.int32)
    srcp = jnp.concatenate([src, padi]).reshape(EPAD // EC, EC)
    dstp = jnp.concatenate([dst, padi]).reshape(EPAD // EC, EC)
    xp = jnp.pad(x, ((0, NP - N), (0, 0)))
    z16 = jnp.zeros((RPW, 8), jnp.float32)
    z128 = jnp.zeros((EC, D), jnp.float32)
    ones16 = jnp.ones((EC, 8), jnp.float32)

    deg2 = _sc_deg(dstp, z16, ones16)
    y1 = _tc1(deg2, xp, W1)
    agg1 = _sc_agg(y1, z128, srcp, dstp)
    y2 = _tc2(deg2, agg1, y1, b1.reshape(1, D), W2)
    agg2 = _sc_agg(y2, z128, srcp, dstp)
    out = _tc3(deg2, agg2, y2, b2.reshape(1, D))
    return out[:N]
